# Initial kernel scaffold; baseline (speedup 1.0000x reference)
import jax
import jax.numpy as jnp
from jax.experimental import pallas as pl


def kernel(x, gate_w, w_gate_up, w_down):
    raise NotImplementedError("write your pallas kernel here")


# speedup vs baseline: 1.1569x; 1.1569x over previous
import functools

import jax
import jax.numpy as jnp
from jax import lax
from jax.experimental import pallas as pl
from jax.experimental.pallas import tpu as pltpu
import jax.experimental.pallas.tpu_sc as plsc

T = 2048
H = 768
E = 8
I = 2048
TM = 128
NT = (T * 2 + E * (TM - 1) + TM - 1) // TM
P = NT * TM
FC = 1024
NCHUNK = I // FC


def _routing_body(x_ref, gw_ref, dest1_ref, dest2_ref, w1_ref, w2_ref,
                  eof_ref, aux_ref):
    x = x_ref[...]
    gw = gw_ref[...]
    logits = lax.dot_general(x, gw, (((1,), (1,)), ((), ())),
                             preferred_element_type=jnp.float32)
    probs = jax.nn.softmax(logits, axis=-1)
    iota_e = lax.broadcasted_iota(jnp.int32, (T, E), 1).astype(jnp.float32)
    big = jnp.float32(1e9)
    m1 = jnp.max(probs, axis=1, keepdims=True)
    e1 = jnp.min(jnp.where(probs == m1, iota_e, big), axis=1, keepdims=True)
    pm = jnp.where(iota_e == e1, -jnp.float32(1.0), probs)
    m2 = jnp.max(pm, axis=1, keepdims=True)
    e2 = jnp.min(jnp.where(pm == m2, iota_e, big), axis=1, keepdims=True)
    s = m1 + m2 + jnp.float32(1e-9)
    w1_ref[...] = (m1 / s).reshape(1, T)
    w2_ref[...] = (m2 / s).reshape(1, T)

    oh1 = (iota_e == e1).astype(jnp.float32)
    oh2 = (iota_e == e2).astype(jnp.float32)
    both = oh1 + oh2
    incl = both
    k = 1
    while k < T:
        incl = incl + jnp.concatenate(
            [jnp.zeros((k, E), jnp.float32), incl[:T - k]], axis=0)
        k *= 2
    excl = incl - both
    cnt = incl[T - 1:T, :]
    pc = jnp.ceil(cnt / TM) * TM
    acc = pc
    k = 1
    while k < E:
        acc = acc + jnp.concatenate(
            [jnp.zeros((1, k), jnp.float32), acc[:, :E - k]], axis=1)
        k *= 2
    offp = acc - pc
    dest1 = jnp.sum(oh1 * (offp + excl), axis=1)
    dest2 = jnp.sum(oh2 * (offp + excl), axis=1)
    dest1_ref[...] = dest1.astype(jnp.int32).reshape(1, T)
    dest2_ref[...] = dest2.astype(jnp.int32).reshape(1, T)

    bnd = (offp + pc)
    jrow = (lax.broadcasted_iota(jnp.int32, (NT, E), 0) * TM).astype(jnp.float32)
    eof = jnp.sum((jrow >= bnd).astype(jnp.int32), axis=1)
    eof_ref[...] = jnp.minimum(eof, E - 1).reshape(1, NT)

    f = cnt / jnp.float32(T)
    pmean = jnp.mean(probs, axis=0, keepdims=True)
    aux_ref[...] = jnp.float32(0.01 * E) * jnp.sum(f * pmean).reshape(1, 1)


def _routing(x_flat, gate_w):
    return pl.pallas_call(
        _routing_body,
        out_shape=(
            jax.ShapeDtypeStruct((1, T), jnp.int32),
            jax.ShapeDtypeStruct((1, T), jnp.int32),
            jax.ShapeDtypeStruct((1, T), jnp.float32),
            jax.ShapeDtypeStruct((1, T), jnp.float32),
            jax.ShapeDtypeStruct((1, NT), jnp.int32),
            jax.ShapeDtypeStruct((1, 1), jnp.float32),
        ),
    )(x_flat, gate_w)


def _mmA_body(eof_ref, x_ref, wg_ref, wu_ref, h_ref):
    x = x_ref[...]
    g = jnp.dot(x, wg_ref[0], preferred_element_type=jnp.float32)
    u = jnp.dot(x, wu_ref[0], preferred_element_type=jnp.float32)
    h_ref[...] = (g * jax.nn.sigmoid(g)) * u


def _mmA(x_sorted, w_gate_up, eof):
    grid_spec = pltpu.PrefetchScalarGridSpec(
        num_scalar_prefetch=1,
        grid=(NCHUNK, NT),
        in_specs=[
            pl.BlockSpec((TM, H), lambda c, j, eof: (j, 0)),
            pl.BlockSpec((1, H, FC), lambda c, j, eof: (eof[j], 0, c)),
            pl.BlockSpec((1, H, FC), lambda c, j, eof: (eof[j], 0, NCHUNK + c)),
        ],
        out_specs=pl.BlockSpec((TM, FC), lambda c, j, eof: (j, c)),
    )
    return pl.pallas_call(
        _mmA_body,
        grid_spec=grid_spec,
        out_shape=jax.ShapeDtypeStruct((P, I), jnp.float32),
    )(eof, x_sorted, w_gate_up, w_gate_up)


def _mmB_body(eof_ref, h_ref, wd_ref, o_ref):
    o_ref[...] = jnp.dot(h_ref[...], wd_ref[0],
                         preferred_element_type=jnp.float32)


def _mmB(h_sorted, w_down, eof):
    grid_spec = pltpu.PrefetchScalarGridSpec(
        num_scalar_prefetch=1,
        grid=(NT,),
        in_specs=[
            pl.BlockSpec((TM, I), lambda j, eof: (j, 0)),
            pl.BlockSpec((1, I, H), lambda j, eof: (eof[j], 0, 0)),
        ],
        out_specs=pl.BlockSpec((TM, H), lambda j, eof: (j, 0)),
    )
    return pl.pallas_call(
        _mmB_body,
        grid_spec=grid_spec,
        out_shape=jax.ShapeDtypeStruct((P, H), jnp.float32),
    )(eof, h_sorted, w_down)


def _build_src(dest1, dest2):
    tok = jnp.arange(T, dtype=jnp.int32)
    src = jnp.zeros((P,), jnp.int32)
    src = src.at[dest1].set(tok).at[dest2].set(tok)
    return src


def _dispatch(x_flat, src_row):
    return x_flat[src_row]


def _combine(os_, dest1, dest2, w1, w2):
    return w1[:, None] * os_[dest1] + w2[:, None] * os_[dest2]


def kernel(x, gate_w, w_gate_up, w_down):
    b, s, h = x.shape
    x_flat = x.reshape(T, H)
    dest1, dest2, w1, w2, eof, aux = _routing(x_flat, gate_w)
    dest1 = dest1.reshape(T)
    dest2 = dest2.reshape(T)
    w1 = w1.reshape(T)
    w2 = w2.reshape(T)
    eof = eof.reshape(NT)
    src_row = _build_src(dest1, dest2)
    x_sorted = _dispatch(x_flat, src_row)
    h_sorted = _mmA(x_sorted, w_gate_up, eof)
    os_ = _mmB(h_sorted, w_down, eof)
    out = _combine(os_, dest1, dest2, w1, w2)
    return out.reshape(b, s, h), aux.reshape(())
